# Initial kernel scaffold; baseline (speedup 1.0000x reference)
import jax
import jax.numpy as jnp
from jax.experimental import pallas as pl


def kernel(node_s, edge_s, edge_index, batch, Wv, bv, We, be, Wm, bm, Wout, bout):
    raise NotImplementedError("write your pallas kernel here")


# speedup vs baseline: 2.4893x; 2.4893x over previous
import functools

import jax
import jax.numpy as jnp
from jax import lax
from jax.experimental import pallas as pl
from jax.experimental.pallas import tpu as pltpu
from jax.experimental.pallas import tpu_sc as plsc

NB = 16
MAXN = 2048
CHUNK = 128


def _ln(x):
    mu = jnp.mean(x, axis=-1, keepdims=True)
    var = jnp.var(x, axis=-1, keepdims=True)
    return (x - mu) * lax.rsqrt(var + 1e-5)


def _node_embed_body(ns_ref, wv_ref, bv_ref, wa_ref, wc_ref,
                     h_ref, p_ref, q_ref):
    h = jnp.dot(_ln(ns_ref[...]), wv_ref[...],
                preferred_element_type=jnp.float32) + bv_ref[...]
    h_ref[...] = h
    p_ref[...] = jnp.dot(h, wa_ref[...], preferred_element_type=jnp.float32)
    q_ref[...] = jnp.dot(h, wc_ref[...], preferred_element_type=jnp.float32)


def _edge_embed_body(es_ref, we_ref, be_ref, wb_ref, bm_ref, r_ref):
    e = jnp.dot(_ln(es_ref[...]), we_ref[...],
                preferred_element_type=jnp.float32) + be_ref[...]
    for l in range(wb_ref.shape[0]):
        r_ref[l] = jnp.dot(e, wb_ref[l],
                           preferred_element_type=jnp.float32) + bm_ref[l]


def _update_body(h_ref, agg_ref, deg_ref, wa_ref, wc_ref,
                 h_ref_out, p_ref, q_ref):
    deg = jnp.maximum(deg_ref[:, 0], 1.0)
    h = _ln(h_ref[...] + agg_ref[...] / deg[:, None])
    h_ref_out[...] = h
    p_ref[...] = jnp.dot(h, wa_ref[...], preferred_element_type=jnp.float32)
    q_ref[...] = jnp.dot(h, wc_ref[...], preferred_element_type=jnp.float32)


def _final_body(h_ref, agg_ref, deg_ref, wo_ref, bo_ref, out_ref):
    deg = jnp.maximum(deg_ref[:, 0], 1.0)
    h = _ln(h_ref[...] + agg_ref[...] / deg[:, None])
    out_ref[...] = jnp.dot(_ln(h), wo_ref[...],
                           preferred_element_type=jnp.float32) + bo_ref[...]


def _pad_body(batch_ref, out_ref, dense_ref):
    b = pl.program_id(0)
    bm = batch_ref[...]
    ptr = jnp.sum(jnp.where(bm < b, 1, 0))
    cnt = jnp.sum(jnp.where(bm == b, 1, 0))
    rows = out_ref[pl.ds(ptr, MAXN), :]
    keep = lax.broadcasted_iota(jnp.int32, (MAXN, 1), 0) < cnt
    dense_ref[0] = jnp.where(keep, rows, 0.0)


def _remap_dst(dstv, c, half):
    lo = c * half
    for j in range(CHUNK // 16):
        sl = pl.ds(j * 16, 16)
        d = dstv[sl] - lo
        ok = (d >= 0) & (d < half)
        dstv[sl] = jnp.where(ok, d, half)


def _sc_deg_body(dst_hbm, zeros_hbm, deg_out, acc, dstv, ones_v, sem):
    c = lax.axis_index("c")
    s = lax.axis_index("s")
    half = acc.shape[0] - 8
    rows_per_tile = half // 16
    for j in range(rows_per_tile // CHUNK):
        pltpu.sync_copy(zeros_hbm,
                        acc.at[pl.ds(s * rows_per_tile + j * CHUNK, CHUNK)])
    one = jnp.ones((16,), jnp.float32)
    for i in range(CHUNK):
        for j in range(8):
            ones_v[i, pl.ds(j * 16, 16)] = one
    plsc.subcore_barrier()

    e = dst_hbm.shape[0]
    per_tile = e // 16
    base0 = s * per_tile

    def body(i, carry):
        base = base0 + i * CHUNK
        pltpu.sync_copy(dst_hbm.at[pl.ds(base, CHUNK)], dstv)
        _remap_dst(dstv, c, half)
        pltpu.sync_copy(ones_v, acc.at[dstv], add=True)
        return carry

    lax.fori_loop(0, per_tile // CHUNK, body, 0)
    plsc.subcore_barrier()
    pltpu.sync_copy(acc.at[pl.ds(s * rows_per_tile, rows_per_tile)],
                    deg_out.at[pl.ds(c * half + s * rows_per_tile,
                                     rows_per_tile)])


def _sc_layer_body(p_hbm, q_hbm, r_hbm, src_hbm, dst_hbm, zeros_hbm,
                   agg_out, acc, srcv, dstv, pbuf, qbuf, rbuf, sem, sem2,
                   *, layer):
    c = lax.axis_index("c")
    s = lax.axis_index("s")
    half = acc.shape[0] - 8
    rows_per_tile = half // 16
    for j in range(rows_per_tile // CHUNK):
        pltpu.sync_copy(zeros_hbm,
                        acc.at[pl.ds(s * rows_per_tile + j * CHUNK, CHUNK)])
    plsc.subcore_barrier()

    e = src_hbm.shape[0]
    per_tile = e // 16
    base0 = s * per_tile

    def body(i, carry):
        base = base0 + i * CHUNK
        pltpu.sync_copy(src_hbm.at[pl.ds(base, CHUNK)], srcv)
        pltpu.sync_copy(dst_hbm.at[pl.ds(base, CHUNK)], dstv)
        cp_p = pltpu.async_copy(p_hbm.at[srcv], pbuf, sem)
        cp_q = pltpu.async_copy(q_hbm.at[dstv], qbuf, sem2)
        pltpu.sync_copy(r_hbm.at[layer].at[pl.ds(base, CHUNK)], rbuf)
        cp_p.wait()
        cp_q.wait()

        def row(rr, cc):
            for j in range(8):
                sl = pl.ds(j * 16, 16)
                m = pbuf[rr, sl] + qbuf[rr, sl] + rbuf[rr, sl]
                pbuf[rr, sl] = jnp.maximum(m, 0.0)
            return cc

        lax.fori_loop(0, CHUNK, row, 0)
        _remap_dst(dstv, c, half)
        pltpu.sync_copy(pbuf, acc.at[dstv], add=True)
        return carry

    lax.fori_loop(0, per_tile // CHUNK, body, 0)
    plsc.subcore_barrier()
    pltpu.sync_copy(acc.at[pl.ds(s * rows_per_tile, rows_per_tile)],
                    agg_out.at[pl.ds(c * half + s * rows_per_tile,
                                     rows_per_tile)])


def kernel(node_s, edge_s, edge_index, batch, Wv, bv, We, be, Wm, bm, Wout, bout):
    n, ns_in = node_s.shape
    e, es_in = edge_s.shape
    layers, msg_in, ns = Wm.shape
    es = We.shape[1]

    src = edge_index[0]
    dst = edge_index[1]
    zeros = jnp.zeros((CHUNK, ns), jnp.float32)

    bn = 1024
    wa = [Wm[l, :ns, :] for l in range(layers)]
    wc = [Wm[l, ns + es:, :] for l in range(layers)]
    wb = Wm[:, ns:ns + es, :]

    h, p, q = pl.pallas_call(
        _node_embed_body,
        grid=(n // bn,),
        in_specs=[
            pl.BlockSpec((bn, ns_in), lambda i: (i, 0)),
            pl.BlockSpec((ns_in, ns), lambda i: (0, 0)),
            pl.BlockSpec((1, ns), lambda i: (0, 0)),
            pl.BlockSpec((ns, ns), lambda i: (0, 0)),
            pl.BlockSpec((ns, ns), lambda i: (0, 0)),
        ],
        out_specs=[
            pl.BlockSpec((bn, ns), lambda i: (i, 0)),
            pl.BlockSpec((bn, ns), lambda i: (i, 0)),
            pl.BlockSpec((bn, ns), lambda i: (i, 0)),
        ],
        out_shape=[
            jax.ShapeDtypeStruct((n, ns), jnp.float32),
            jax.ShapeDtypeStruct((n, ns), jnp.float32),
            jax.ShapeDtypeStruct((n, ns), jnp.float32),
        ],
    )(node_s, Wv, bv.reshape(1, ns), wa[0], wc[0])

    be_blk = 2048
    r = pl.pallas_call(
        _edge_embed_body,
        grid=(e // be_blk,),
        in_specs=[
            pl.BlockSpec((be_blk, es_in), lambda i: (i, 0)),
            pl.BlockSpec((es_in, es), lambda i: (0, 0)),
            pl.BlockSpec((1, es), lambda i: (0, 0)),
            pl.BlockSpec((layers, es, ns), lambda i: (0, 0, 0)),
            pl.BlockSpec((layers, 1, ns), lambda i: (0, 0, 0)),
        ],
        out_specs=pl.BlockSpec((layers, be_blk, ns), lambda i: (0, i, 0)),
        out_shape=jax.ShapeDtypeStruct((layers, e, ns), jnp.float32),
    )(edge_s, We, be.reshape(1, es), wb, bm.reshape(layers, 1, ns))

    mesh = plsc.VectorSubcoreMesh(core_axis_name="c", subcore_axis_name="s")
    deg = pl.kernel(
        _sc_deg_body,
        out_type=jax.ShapeDtypeStruct((n, ns), jnp.float32),
        mesh=mesh,
        scratch_types=[
            pltpu.VMEM_SHARED((n // 2 + 8, ns), jnp.float32),
            pltpu.VMEM((CHUNK,), jnp.int32),
            pltpu.VMEM((CHUNK, ns), jnp.float32),
            pltpu.SemaphoreType.DMA,
        ],
    )(dst, zeros)

    for l in range(layers):
        agg = pl.kernel(
            functools.partial(_sc_layer_body, layer=l),
            out_type=jax.ShapeDtypeStruct((n, ns), jnp.float32),
            mesh=mesh,
            scratch_types=[
                pltpu.VMEM_SHARED((n // 2 + 8, ns), jnp.float32),
                pltpu.VMEM((CHUNK,), jnp.int32),
                pltpu.VMEM((CHUNK,), jnp.int32),
                pltpu.VMEM((CHUNK, ns), jnp.float32),
                pltpu.VMEM((CHUNK, ns), jnp.float32),
                pltpu.VMEM((CHUNK, ns), jnp.float32),
                pltpu.SemaphoreType.DMA,
                pltpu.SemaphoreType.DMA,
            ],
        )(p, q, r, src, dst, zeros)

        if l < layers - 1:
            h, p, q = pl.pallas_call(
                _update_body,
                grid=(n // bn,),
                in_specs=[
                    pl.BlockSpec((bn, ns), lambda i: (i, 0)),
                    pl.BlockSpec((bn, ns), lambda i: (i, 0)),
                    pl.BlockSpec((bn, ns), lambda i: (i, 0)),
                    pl.BlockSpec((ns, ns), lambda i: (0, 0)),
                    pl.BlockSpec((ns, ns), lambda i: (0, 0)),
                ],
                out_specs=[
                    pl.BlockSpec((bn, ns), lambda i: (i, 0)),
                    pl.BlockSpec((bn, ns), lambda i: (i, 0)),
                    pl.BlockSpec((bn, ns), lambda i: (i, 0)),
                ],
                out_shape=[
                    jax.ShapeDtypeStruct((n, ns), jnp.float32),
                    jax.ShapeDtypeStruct((n, ns), jnp.float32),
                    jax.ShapeDtypeStruct((n, ns), jnp.float32),
                ],
            )(h, agg, deg, wa[l + 1], wc[l + 1])
        else:
            out = pl.pallas_call(
                _final_body,
                grid=(n // bn,),
                in_specs=[
                    pl.BlockSpec((bn, ns), lambda i: (i, 0)),
                    pl.BlockSpec((bn, ns), lambda i: (i, 0)),
                    pl.BlockSpec((bn, ns), lambda i: (i, 0)),
                    pl.BlockSpec((ns, ns), lambda i: (0, 0)),
                    pl.BlockSpec((1, ns), lambda i: (0, 0)),
                ],
                out_specs=pl.BlockSpec((bn, ns), lambda i: (i, 0)),
                out_shape=jax.ShapeDtypeStruct((n, ns), jnp.float32),
            )(h, agg, deg, Wout, bout.reshape(1, ns))

    outp = jnp.pad(out, ((0, MAXN), (0, 0)))
    dense = pl.pallas_call(
        _pad_body,
        grid=(NB,),
        in_specs=[
            pl.BlockSpec((n // 128, 128), lambda b: (0, 0)),
            pl.BlockSpec((n + MAXN, ns), lambda b: (0, 0)),
        ],
        out_specs=pl.BlockSpec((1, MAXN, ns), lambda b: (b, 0, 0)),
        out_shape=jax.ShapeDtypeStruct((NB, MAXN, ns), jnp.float32),
    )(batch.reshape(n // 128, 128), outp)
    return dense


# speedup vs baseline: 2.7615x; 1.1094x over previous
import functools

import jax
import jax.numpy as jnp
from jax import lax
from jax.experimental import pallas as pl
from jax.experimental.pallas import tpu as pltpu
from jax.experimental.pallas import tpu_sc as plsc

NB = 16
MAXN = 2048
CHUNK = 64


def _ln(x):
    mu = jnp.mean(x, axis=-1, keepdims=True)
    var = jnp.var(x, axis=-1, keepdims=True)
    return (x - mu) * lax.rsqrt(var + 1e-5)


def _node_embed_body(ns_ref, wv_ref, bv_ref, wa_ref, wc_ref,
                     h_ref, p_ref, q_ref):
    h = jnp.dot(_ln(ns_ref[...]), wv_ref[...],
                preferred_element_type=jnp.float32) + bv_ref[...]
    h_ref[...] = h
    p_ref[...] = jnp.dot(h, wa_ref[...], preferred_element_type=jnp.float32)
    q_ref[...] = jnp.dot(h, wc_ref[...], preferred_element_type=jnp.float32)


def _edge_embed_body(es_ref, we_ref, be_ref, wb_ref, bm_ref, r_ref):
    e = jnp.dot(_ln(es_ref[...]), we_ref[...],
                preferred_element_type=jnp.float32) + be_ref[...]
    for l in range(wb_ref.shape[0]):
        r_ref[l] = jnp.dot(e, wb_ref[l],
                           preferred_element_type=jnp.float32) + bm_ref[l]


def _update_body(h_ref, agg_ref, deg_ref, wa_ref, wc_ref,
                 h_ref_out, p_ref, q_ref):
    deg = jnp.maximum(deg_ref[:, 0], 1.0)
    h = _ln(h_ref[...] + agg_ref[...] / deg[:, None])
    h_ref_out[...] = h
    p_ref[...] = jnp.dot(h, wa_ref[...], preferred_element_type=jnp.float32)
    q_ref[...] = jnp.dot(h, wc_ref[...], preferred_element_type=jnp.float32)


def _final_body(h_ref, agg_ref, deg_ref, wo_ref, bo_ref, out_ref):
    deg = jnp.maximum(deg_ref[:, 0], 1.0)
    h = _ln(h_ref[...] + agg_ref[...] / deg[:, None])
    out_ref[...] = jnp.dot(_ln(h), wo_ref[...],
                           preferred_element_type=jnp.float32) + bo_ref[...]


def _pad_body(batch_ref, out_ref, dense_ref):
    b = pl.program_id(0)
    bm = batch_ref[...]
    ptr = jnp.sum(jnp.where(bm < b, 1, 0))
    cnt = jnp.sum(jnp.where(bm == b, 1, 0))
    rows = out_ref[pl.ds(ptr, MAXN), :]
    keep = lax.broadcasted_iota(jnp.int32, (MAXN, 1), 0) < cnt
    dense_ref[0] = jnp.where(keep, rows, 0.0)


def _remap_dst(dstv, c, half):
    lo = c * half
    for j in range(CHUNK // 16):
        sl = pl.ds(j * 16, 16)
        d = dstv[sl] - lo
        ok = (d >= 0) & (d < half)
        dstv[sl] = jnp.where(ok, d, half)


def _sc_deg_body(dst_hbm, zeros_hbm, deg_out, acc, dstv, ones_v, sem):
    c = lax.axis_index("c")
    s = lax.axis_index("s")
    half = acc.shape[0] - 8
    rows_per_tile = half // 16
    for j in range(rows_per_tile // CHUNK):
        pltpu.sync_copy(zeros_hbm,
                        acc.at[pl.ds(s * rows_per_tile + j * CHUNK, CHUNK)])
    one = jnp.ones((16,), jnp.float32)
    for i in range(CHUNK):
        for j in range(8):
            ones_v[i, pl.ds(j * 16, 16)] = one
    plsc.subcore_barrier()

    e = dst_hbm.shape[0]
    per_tile = e // 16
    base0 = s * per_tile

    def body(i, carry):
        base = base0 + i * CHUNK
        pltpu.sync_copy(dst_hbm.at[pl.ds(base, CHUNK)], dstv)
        _remap_dst(dstv, c, half)
        pltpu.sync_copy(ones_v, acc.at[dstv], add=True)
        return carry

    lax.fori_loop(0, per_tile // CHUNK, body, 0)
    plsc.subcore_barrier()
    pltpu.sync_copy(acc.at[pl.ds(s * rows_per_tile, rows_per_tile)],
                    deg_out.at[pl.ds(c * half + s * rows_per_tile,
                                     rows_per_tile)])


def _sc_layer_body(p_hbm, q_hbm, r_hbm, src_hbm, dst_hbm, zeros_hbm,
                   agg_out, acc,
                   srcv0, dstv0, dstl0, pbuf0, qbuf0, rbuf0,
                   srcv1, dstv1, dstl1, pbuf1, qbuf1, rbuf1,
                   gsem0, gsem1, *, layer):
    c = lax.axis_index("c")
    s = lax.axis_index("s")
    half = acc.shape[0] - 8
    rows_per_tile = half // 16
    for j in range(rows_per_tile // CHUNK):
        pltpu.sync_copy(zeros_hbm,
                        acc.at[pl.ds(s * rows_per_tile + j * CHUNK, CHUNK)])
    plsc.subcore_barrier()

    e = src_hbm.shape[0]
    per_tile = e // 16
    nchunks = per_tile // CHUNK
    base0 = s * per_tile

    bufs = ((srcv0, dstv0, dstl0, pbuf0, qbuf0, rbuf0, gsem0),
            (srcv1, dstv1, dstl1, pbuf1, qbuf1, rbuf1, gsem1))

    def issue(g, bb):
        srcv, dstv, _, pbuf, qbuf, rbuf, gsem = bb
        base = base0 + g * CHUNK
        pltpu.sync_copy(src_hbm.at[pl.ds(base, CHUNK)], srcv)
        pltpu.sync_copy(dst_hbm.at[pl.ds(base, CHUNK)], dstv)
        pltpu.async_copy(p_hbm.at[srcv], pbuf, gsem)
        pltpu.async_copy(q_hbm.at[dstv], qbuf, gsem)
        pltpu.async_copy(r_hbm.at[layer].at[pl.ds(base, CHUNK)], rbuf, gsem)

    def drain(bb):
        srcv, dstv, _, pbuf, qbuf, rbuf, gsem = bb
        pltpu.make_async_copy(p_hbm.at[srcv], pbuf, gsem).wait()
        pltpu.make_async_copy(q_hbm.at[dstv], qbuf, gsem).wait()
        pltpu.make_async_copy(r_hbm.at[0].at[pl.ds(0, CHUNK)], rbuf, gsem).wait()

    issue(0, bufs[0])

    @pl.loop(0, nchunks // 2)
    def _outer(gg):
        for b in range(2):
            srcv, dstv, dstl, pbuf, qbuf, rbuf, gsem = bufs[b]
            g = gg * 2 + b
            drain(bufs[b])

            @pl.when(g + 1 < nchunks)
            def _():
                issue(g + 1, bufs[1 - b])

            @plsc.parallel_loop(0, CHUNK, unroll=4)
            def _rows(rr):
                for j in range(8):
                    sl = pl.ds(j * 16, 16)
                    m = pbuf[rr, sl] + qbuf[rr, sl] + rbuf[rr, sl]
                    pbuf[rr, sl] = jnp.maximum(m, 0.0)

            lo = c * half
            for j in range(CHUNK // 16):
                sl = pl.ds(j * 16, 16)
                d = dstv[sl] - lo
                ok = (d >= 0) & (d < half)
                dstl[sl] = jnp.where(ok, d, half)
            pltpu.sync_copy(pbuf, acc.at[dstl], add=True)

    plsc.subcore_barrier()
    pltpu.sync_copy(acc.at[pl.ds(s * rows_per_tile, rows_per_tile)],
                    agg_out.at[pl.ds(c * half + s * rows_per_tile,
                                     rows_per_tile)])


def kernel(node_s, edge_s, edge_index, batch, Wv, bv, We, be, Wm, bm, Wout, bout):
    n, ns_in = node_s.shape
    e, es_in = edge_s.shape
    layers, msg_in, ns = Wm.shape
    es = We.shape[1]

    src = edge_index[0]
    dst = edge_index[1]
    zeros = jnp.zeros((CHUNK, ns), jnp.float32)

    bn = 1024
    wa = [Wm[l, :ns, :] for l in range(layers)]
    wc = [Wm[l, ns + es:, :] for l in range(layers)]
    wb = Wm[:, ns:ns + es, :]

    h, p, q = pl.pallas_call(
        _node_embed_body,
        grid=(n // bn,),
        in_specs=[
            pl.BlockSpec((bn, ns_in), lambda i: (i, 0)),
            pl.BlockSpec((ns_in, ns), lambda i: (0, 0)),
            pl.BlockSpec((1, ns), lambda i: (0, 0)),
            pl.BlockSpec((ns, ns), lambda i: (0, 0)),
            pl.BlockSpec((ns, ns), lambda i: (0, 0)),
        ],
        out_specs=[
            pl.BlockSpec((bn, ns), lambda i: (i, 0)),
            pl.BlockSpec((bn, ns), lambda i: (i, 0)),
            pl.BlockSpec((bn, ns), lambda i: (i, 0)),
        ],
        out_shape=[
            jax.ShapeDtypeStruct((n, ns), jnp.float32),
            jax.ShapeDtypeStruct((n, ns), jnp.float32),
            jax.ShapeDtypeStruct((n, ns), jnp.float32),
        ],
    )(node_s, Wv, bv.reshape(1, ns), wa[0], wc[0])

    be_blk = 2048
    r = pl.pallas_call(
        _edge_embed_body,
        grid=(e // be_blk,),
        in_specs=[
            pl.BlockSpec((be_blk, es_in), lambda i: (i, 0)),
            pl.BlockSpec((es_in, es), lambda i: (0, 0)),
            pl.BlockSpec((1, es), lambda i: (0, 0)),
            pl.BlockSpec((layers, es, ns), lambda i: (0, 0, 0)),
            pl.BlockSpec((layers, 1, ns), lambda i: (0, 0, 0)),
        ],
        out_specs=pl.BlockSpec((layers, be_blk, ns), lambda i: (0, i, 0)),
        out_shape=jax.ShapeDtypeStruct((layers, e, ns), jnp.float32),
    )(edge_s, We, be.reshape(1, es), wb, bm.reshape(layers, 1, ns))

    mesh = plsc.VectorSubcoreMesh(core_axis_name="c", subcore_axis_name="s")
    deg = pl.kernel(
        _sc_deg_body,
        out_type=jax.ShapeDtypeStruct((n, ns), jnp.float32),
        mesh=mesh,
        scratch_types=[
            pltpu.VMEM_SHARED((n // 2 + 8, ns), jnp.float32),
            pltpu.VMEM((CHUNK,), jnp.int32),
            pltpu.VMEM((CHUNK, ns), jnp.float32),
            pltpu.SemaphoreType.DMA,
        ],
    )(dst, zeros)

    for l in range(layers):
        agg = pl.kernel(
            functools.partial(_sc_layer_body, layer=l),
            out_type=jax.ShapeDtypeStruct((n, ns), jnp.float32),
            mesh=mesh,
            scratch_types=[
                pltpu.VMEM_SHARED((n // 2 + 8, ns), jnp.float32),
                pltpu.VMEM((CHUNK,), jnp.int32),
                pltpu.VMEM((CHUNK,), jnp.int32),
                pltpu.VMEM((CHUNK,), jnp.int32),
                pltpu.VMEM((CHUNK, ns), jnp.float32),
                pltpu.VMEM((CHUNK, ns), jnp.float32),
                pltpu.VMEM((CHUNK, ns), jnp.float32),
                pltpu.VMEM((CHUNK,), jnp.int32),
                pltpu.VMEM((CHUNK,), jnp.int32),
                pltpu.VMEM((CHUNK,), jnp.int32),
                pltpu.VMEM((CHUNK, ns), jnp.float32),
                pltpu.VMEM((CHUNK, ns), jnp.float32),
                pltpu.VMEM((CHUNK, ns), jnp.float32),
                pltpu.SemaphoreType.DMA,
                pltpu.SemaphoreType.DMA,
            ],
        )(p, q, r, src, dst, zeros)

        if l < layers - 1:
            h, p, q = pl.pallas_call(
                _update_body,
                grid=(n // bn,),
                in_specs=[
                    pl.BlockSpec((bn, ns), lambda i: (i, 0)),
                    pl.BlockSpec((bn, ns), lambda i: (i, 0)),
                    pl.BlockSpec((bn, ns), lambda i: (i, 0)),
                    pl.BlockSpec((ns, ns), lambda i: (0, 0)),
                    pl.BlockSpec((ns, ns), lambda i: (0, 0)),
                ],
                out_specs=[
                    pl.BlockSpec((bn, ns), lambda i: (i, 0)),
                    pl.BlockSpec((bn, ns), lambda i: (i, 0)),
                    pl.BlockSpec((bn, ns), lambda i: (i, 0)),
                ],
                out_shape=[
                    jax.ShapeDtypeStruct((n, ns), jnp.float32),
                    jax.ShapeDtypeStruct((n, ns), jnp.float32),
                    jax.ShapeDtypeStruct((n, ns), jnp.float32),
                ],
            )(h, agg, deg, wa[l + 1], wc[l + 1])
        else:
            out = pl.pallas_call(
                _final_body,
                grid=(n // bn,),
                in_specs=[
                    pl.BlockSpec((bn, ns), lambda i: (i, 0)),
                    pl.BlockSpec((bn, ns), lambda i: (i, 0)),
                    pl.BlockSpec((bn, ns), lambda i: (i, 0)),
                    pl.BlockSpec((ns, ns), lambda i: (0, 0)),
                    pl.BlockSpec((1, ns), lambda i: (0, 0)),
                ],
                out_specs=pl.BlockSpec((bn, ns), lambda i: (i, 0)),
                out_shape=jax.ShapeDtypeStruct((n, ns), jnp.float32),
            )(h, agg, deg, Wout, bout.reshape(1, ns))

    outp = jnp.pad(out, ((0, MAXN), (0, 0)))
    dense = pl.pallas_call(
        _pad_body,
        grid=(NB,),
        in_specs=[
            pl.BlockSpec((n // 128, 128), lambda b: (0, 0)),
            pl.BlockSpec((n + MAXN, ns), lambda b: (0, 0)),
        ],
        out_specs=pl.BlockSpec((1, MAXN, ns), lambda b: (b, 0, 0)),
        out_shape=jax.ShapeDtypeStruct((NB, MAXN, ns), jnp.float32),
    )(batch.reshape(n // 128, 128), outp)
    return dense


# speedup vs baseline: 3.1718x; 1.1486x over previous
import functools

import jax
import jax.numpy as jnp
from jax import lax
from jax.experimental import pallas as pl
from jax.experimental.pallas import tpu as pltpu
from jax.experimental.pallas import tpu_sc as plsc

NB = 16
MAXN = 2048
CHUNK = 64


def _ln(x):
    mu = jnp.mean(x, axis=-1, keepdims=True)
    var = jnp.var(x, axis=-1, keepdims=True)
    return (x - mu) * lax.rsqrt(var + 1e-5)


def _node_embed_body(ns_ref, wv_ref, bv_ref, wa_ref, wc_ref,
                     h_ref, p_ref, q_ref):
    h = jnp.dot(_ln(ns_ref[...]), wv_ref[...],
                preferred_element_type=jnp.float32) + bv_ref[...]
    h_ref[...] = h
    p_ref[...] = jnp.dot(h, wa_ref[...], preferred_element_type=jnp.float32)
    q_ref[...] = jnp.dot(h, wc_ref[...], preferred_element_type=jnp.float32)


def _edge_embed_body(es_ref, we_ref, be_ref, wb_ref, bm_ref, r_ref):
    e = jnp.dot(_ln(es_ref[...]), we_ref[...],
                preferred_element_type=jnp.float32) + be_ref[...]
    for l in range(wb_ref.shape[0]):
        r_ref[l] = jnp.dot(e, wb_ref[l],
                           preferred_element_type=jnp.float32) + bm_ref[l]


def _update_body(h_ref, agg_ref, deg_ref, wa_ref, wc_ref,
                 h_ref_out, p_ref, q_ref):
    deg = jnp.maximum(deg_ref[:, 0], 1.0)
    h = _ln(h_ref[...] + agg_ref[...] / deg[:, None])
    h_ref_out[...] = h
    p_ref[...] = jnp.dot(h, wa_ref[...], preferred_element_type=jnp.float32)
    q_ref[...] = jnp.dot(h, wc_ref[...], preferred_element_type=jnp.float32)


def _final_body(h_ref, agg_ref, deg_ref, wo_ref, bo_ref, out_ref):
    deg = jnp.maximum(deg_ref[:, 0], 1.0)
    h = _ln(h_ref[...] + agg_ref[...] / deg[:, None])
    out_ref[...] = jnp.dot(_ln(h), wo_ref[...],
                           preferred_element_type=jnp.float32) + bo_ref[...]


def _pad_body(batch_ref, out_ref, dense_ref):
    b = pl.program_id(0)
    bm = batch_ref[...]
    ptr = jnp.sum(jnp.where(bm < b, 1, 0))
    cnt = jnp.sum(jnp.where(bm == b, 1, 0))
    rows = out_ref[pl.ds(ptr, MAXN), :]
    keep = lax.broadcasted_iota(jnp.int32, (MAXN, 1), 0) < cnt
    dense_ref[0] = jnp.where(keep, rows, 0.0)


def _sc_prep_body(src_hbm, dst_hbm, zeros_hbm,
                  src_c, dst_c, eid_c, cnts, deg_out,
                  acc, sv, dv, dlv, sbuf, dbuf, ebuf, ones_v, cbuf, sem):
    c = lax.axis_index("c")
    s = lax.axis_index("s")
    half = acc.shape[0] - 8
    rows_per_tile = half // 16
    for j in range(rows_per_tile // CHUNK):
        pltpu.sync_copy(zeros_hbm,
                        acc.at[pl.ds(s * rows_per_tile + j * CHUNK, CHUNK)])
    one = jnp.ones((16,), jnp.float32)
    zero_i = jnp.zeros((16,), jnp.int32)
    for i in range(CHUNK):
        for j in range(8):
            ones_v[i, pl.ds(j * 16, 16)] = one
    trash_dst = jnp.full((16,), 0, jnp.int32) + (1 - c) * half
    per_tile = src_hbm.shape[0] // 16
    for i in range(per_tile // 16):
        sl = pl.ds(i * 16, 16)
        sbuf[sl] = zero_i
        dbuf[sl] = trash_dst
        ebuf[sl] = zero_i
    plsc.subcore_barrier()

    base0 = s * per_tile
    lo = c * half

    def body(g, off):
        base = base0 + g * CHUNK
        pltpu.sync_copy(src_hbm.at[pl.ds(base, CHUNK)], sv)
        pltpu.sync_copy(dst_hbm.at[pl.ds(base, CHUNK)], dv)
        for j in range(CHUNK // 16):
            sl = pl.ds(j * 16, 16)
            d = dv[sl]
            dl = d - lo
            ok = (dl >= 0) & (dl < half)
            dlv[sl] = jnp.where(ok, dl, half)
            eid = base + j * 16 + lax.iota(jnp.int32, 16)
            ok_i = ok.astype(jnp.int32)
            pos = plsc.cumsum(ok_i) - 1 + off
            idx = jnp.where(ok, pos, per_tile)
            plsc.store_scatter(sbuf, [idx], sv[sl])
            plsc.store_scatter(dbuf, [idx], d)
            plsc.store_scatter(ebuf, [idx], eid)
            off = off + jnp.sum(ok_i)
        pltpu.sync_copy(ones_v, acc.at[dlv], add=True)
        return off

    n_valid = lax.fori_loop(0, per_tile // CHUNK, body, 0)

    flat0 = c * src_hbm.shape[0] + base0
    pltpu.sync_copy(sbuf.at[pl.ds(0, per_tile)], src_c.at[pl.ds(flat0, per_tile)])
    pltpu.sync_copy(dbuf.at[pl.ds(0, per_tile)], dst_c.at[pl.ds(flat0, per_tile)])
    pltpu.sync_copy(ebuf.at[pl.ds(0, per_tile)], eid_c.at[pl.ds(flat0, per_tile)])
    cbuf[...] = jnp.zeros((16,), jnp.int32) + n_valid
    pltpu.sync_copy(cbuf, cnts.at[pl.ds(c * 256 + s * 16, 16)])

    plsc.subcore_barrier()
    pltpu.sync_copy(acc.at[pl.ds(s * rows_per_tile, rows_per_tile)],
                    deg_out.at[pl.ds(c * half + s * rows_per_tile,
                                     rows_per_tile)])


def _sc_layer_body(p_hbm, q_hbm, r_hbm, src_c, dst_c, eid_c, cnts, zeros_hbm,
                   agg_out, acc,
                   srcv0, dstv0, dstl0, eidv0, pbuf0, qbuf0, rbuf0,
                   srcv1, dstv1, dstl1, eidv1, pbuf1, qbuf1, rbuf1,
                   cntv, gsem0, gsem1, *, layer):
    c = lax.axis_index("c")
    s = lax.axis_index("s")
    half = acc.shape[0] - 8
    rows_per_tile = half // 16
    for j in range(rows_per_tile // CHUNK):
        pltpu.sync_copy(zeros_hbm,
                        acc.at[pl.ds(s * rows_per_tile + j * CHUNK, CHUNK)])
    pltpu.sync_copy(cnts.at[pl.ds(c * 256 + s * 16, 16)], cntv)
    plsc.subcore_barrier()

    e = src_c.shape[0] // 2
    per_tile = e // 16
    n_valid = jnp.max(cntv[...])
    nch = (n_valid + CHUNK - 1) // CHUNK
    flat0 = c * e + s * per_tile

    bufs = ((srcv0, dstv0, dstl0, eidv0, pbuf0, qbuf0, rbuf0, gsem0),
            (srcv1, dstv1, dstl1, eidv1, pbuf1, qbuf1, rbuf1, gsem1))

    def issue(g, bb):
        srcv, dstv, _, eidv, pbuf, qbuf, rbuf, gsem = bb
        base = flat0 + g * CHUNK
        pltpu.sync_copy(src_c.at[pl.ds(base, CHUNK)], srcv)
        pltpu.sync_copy(dst_c.at[pl.ds(base, CHUNK)], dstv)
        pltpu.sync_copy(eid_c.at[pl.ds(base, CHUNK)], eidv)
        pltpu.async_copy(p_hbm.at[srcv], pbuf, gsem)
        pltpu.async_copy(q_hbm.at[dstv], qbuf, gsem)
        pltpu.async_copy(r_hbm.at[layer].at[eidv], rbuf, gsem)

    def drain(bb):
        srcv, dstv, _, eidv, pbuf, qbuf, rbuf, gsem = bb
        pltpu.make_async_copy(p_hbm.at[srcv], pbuf, gsem).wait()
        pltpu.make_async_copy(q_hbm.at[dstv], qbuf, gsem).wait()
        pltpu.make_async_copy(p_hbm.at[eidv], rbuf, gsem).wait()

    @pl.when(0 < nch)
    def _():
        issue(0, bufs[0])

    @pl.loop(0, (nch + 1) // 2)
    def _outer(gg):
        for b in range(2):
            srcv, dstv, dstl, eidv, pbuf, qbuf, rbuf, gsem = bufs[b]
            g = gg * 2 + b

            @pl.when(g < nch)
            def _():
                drain(bufs[b])

                @pl.when(g + 1 < nch)
                def _():
                    issue(g + 1, bufs[1 - b])

                @plsc.parallel_loop(0, CHUNK, unroll=4)
                def _rows(rr):
                    for j in range(8):
                        sl = pl.ds(j * 16, 16)
                        m = pbuf[rr, sl] + qbuf[rr, sl] + rbuf[rr, sl]
                        pbuf[rr, sl] = jnp.maximum(m, 0.0)

                lo = c * half
                for j in range(CHUNK // 16):
                    sl = pl.ds(j * 16, 16)
                    d = dstv[sl] - lo
                    ok = (d >= 0) & (d < half)
                    dstl[sl] = jnp.where(ok, d, half)
                pltpu.sync_copy(pbuf, acc.at[dstl], add=True)

    plsc.subcore_barrier()
    pltpu.sync_copy(acc.at[pl.ds(s * rows_per_tile, rows_per_tile)],
                    agg_out.at[pl.ds(c * half + s * rows_per_tile,
                                     rows_per_tile)])


def kernel(node_s, edge_s, edge_index, batch, Wv, bv, We, be, Wm, bm, Wout, bout):
    n, ns_in = node_s.shape
    e, es_in = edge_s.shape
    layers, msg_in, ns = Wm.shape
    es = We.shape[1]

    src = edge_index[0]
    dst = edge_index[1]
    zeros = jnp.zeros((CHUNK, ns), jnp.float32)

    bn = 1024
    wa = [Wm[l, :ns, :] for l in range(layers)]
    wc = [Wm[l, ns + es:, :] for l in range(layers)]
    wb = Wm[:, ns:ns + es, :]

    h, p, q = pl.pallas_call(
        _node_embed_body,
        grid=(n // bn,),
        in_specs=[
            pl.BlockSpec((bn, ns_in), lambda i: (i, 0)),
            pl.BlockSpec((ns_in, ns), lambda i: (0, 0)),
            pl.BlockSpec((1, ns), lambda i: (0, 0)),
            pl.BlockSpec((ns, ns), lambda i: (0, 0)),
            pl.BlockSpec((ns, ns), lambda i: (0, 0)),
        ],
        out_specs=[
            pl.BlockSpec((bn, ns), lambda i: (i, 0)),
            pl.BlockSpec((bn, ns), lambda i: (i, 0)),
            pl.BlockSpec((bn, ns), lambda i: (i, 0)),
        ],
        out_shape=[
            jax.ShapeDtypeStruct((n, ns), jnp.float32),
            jax.ShapeDtypeStruct((n, ns), jnp.float32),
            jax.ShapeDtypeStruct((n, ns), jnp.float32),
        ],
    )(node_s, Wv, bv.reshape(1, ns), wa[0], wc[0])

    be_blk = 2048
    r = pl.pallas_call(
        _edge_embed_body,
        grid=(e // be_blk,),
        in_specs=[
            pl.BlockSpec((be_blk, es_in), lambda i: (i, 0)),
            pl.BlockSpec((es_in, es), lambda i: (0, 0)),
            pl.BlockSpec((1, es), lambda i: (0, 0)),
            pl.BlockSpec((layers, es, ns), lambda i: (0, 0, 0)),
            pl.BlockSpec((layers, 1, ns), lambda i: (0, 0, 0)),
        ],
        out_specs=pl.BlockSpec((layers, be_blk, ns), lambda i: (0, i, 0)),
        out_shape=jax.ShapeDtypeStruct((layers, e, ns), jnp.float32),
    )(edge_s, We, be.reshape(1, es), wb, bm.reshape(layers, 1, ns))

    mesh = plsc.VectorSubcoreMesh(core_axis_name="c", subcore_axis_name="s")
    per_tile = e // 16
    src_c, dst_c, eid_c, cnts, deg = pl.kernel(
        _sc_prep_body,
        out_type=[
            jax.ShapeDtypeStruct((2 * e,), jnp.int32),
            jax.ShapeDtypeStruct((2 * e,), jnp.int32),
            jax.ShapeDtypeStruct((2 * e,), jnp.int32),
            jax.ShapeDtypeStruct((512,), jnp.int32),
            jax.ShapeDtypeStruct((n, ns), jnp.float32),
        ],
        mesh=mesh,
        compiler_params=pltpu.CompilerParams(needs_layout_passes=False),
        scratch_types=[
            pltpu.VMEM_SHARED((n // 2 + 8, ns), jnp.float32),
            pltpu.VMEM((CHUNK,), jnp.int32),
            pltpu.VMEM((CHUNK,), jnp.int32),
            pltpu.VMEM((CHUNK,), jnp.int32),
            pltpu.VMEM((per_tile + 16,), jnp.int32),
            pltpu.VMEM((per_tile + 16,), jnp.int32),
            pltpu.VMEM((per_tile + 16,), jnp.int32),
            pltpu.VMEM((CHUNK, ns), jnp.float32),
            pltpu.VMEM((16,), jnp.int32),
            pltpu.SemaphoreType.DMA,
        ],
    )(src, dst, zeros)

    for l in range(layers):
        agg = pl.kernel(
            functools.partial(_sc_layer_body, layer=l),
            out_type=jax.ShapeDtypeStruct((n, ns), jnp.float32),
            mesh=mesh,
            compiler_params=pltpu.CompilerParams(needs_layout_passes=False),
            scratch_types=[
                pltpu.VMEM_SHARED((n // 2 + 8, ns), jnp.float32),
                pltpu.VMEM((CHUNK,), jnp.int32),
                pltpu.VMEM((CHUNK,), jnp.int32),
                pltpu.VMEM((CHUNK,), jnp.int32),
                pltpu.VMEM((CHUNK,), jnp.int32),
                pltpu.VMEM((CHUNK, ns), jnp.float32),
                pltpu.VMEM((CHUNK, ns), jnp.float32),
                pltpu.VMEM((CHUNK, ns), jnp.float32),
                pltpu.VMEM((CHUNK,), jnp.int32),
                pltpu.VMEM((CHUNK,), jnp.int32),
                pltpu.VMEM((CHUNK,), jnp.int32),
                pltpu.VMEM((CHUNK,), jnp.int32),
                pltpu.VMEM((CHUNK, ns), jnp.float32),
                pltpu.VMEM((CHUNK, ns), jnp.float32),
                pltpu.VMEM((CHUNK, ns), jnp.float32),
                pltpu.VMEM((16,), jnp.int32),
                pltpu.SemaphoreType.DMA,
                pltpu.SemaphoreType.DMA,
            ],
        )(p, q, r, src_c, dst_c, eid_c, cnts, zeros)

        if l < layers - 1:
            h, p, q = pl.pallas_call(
                _update_body,
                grid=(n // bn,),
                in_specs=[
                    pl.BlockSpec((bn, ns), lambda i: (i, 0)),
                    pl.BlockSpec((bn, ns), lambda i: (i, 0)),
                    pl.BlockSpec((bn, ns), lambda i: (i, 0)),
                    pl.BlockSpec((ns, ns), lambda i: (0, 0)),
                    pl.BlockSpec((ns, ns), lambda i: (0, 0)),
                ],
                out_specs=[
                    pl.BlockSpec((bn, ns), lambda i: (i, 0)),
                    pl.BlockSpec((bn, ns), lambda i: (i, 0)),
                    pl.BlockSpec((bn, ns), lambda i: (i, 0)),
                ],
                out_shape=[
                    jax.ShapeDtypeStruct((n, ns), jnp.float32),
                    jax.ShapeDtypeStruct((n, ns), jnp.float32),
                    jax.ShapeDtypeStruct((n, ns), jnp.float32),
                ],
            )(h, agg, deg, wa[l + 1], wc[l + 1])
        else:
            out = pl.pallas_call(
                _final_body,
                grid=(n // bn,),
                in_specs=[
                    pl.BlockSpec((bn, ns), lambda i: (i, 0)),
                    pl.BlockSpec((bn, ns), lambda i: (i, 0)),
                    pl.BlockSpec((bn, ns), lambda i: (i, 0)),
                    pl.BlockSpec((ns, ns), lambda i: (0, 0)),
                    pl.BlockSpec((1, ns), lambda i: (0, 0)),
                ],
                out_specs=pl.BlockSpec((bn, ns), lambda i: (i, 0)),
                out_shape=jax.ShapeDtypeStruct((n, ns), jnp.float32),
            )(h, agg, deg, Wout, bout.reshape(1, ns))

    outp = jnp.pad(out, ((0, MAXN), (0, 0)))
    dense = pl.pallas_call(
        _pad_body,
        grid=(NB,),
        in_specs=[
            pl.BlockSpec((n // 128, 128), lambda b: (0, 0)),
            pl.BlockSpec((n + MAXN, ns), lambda b: (0, 0)),
        ],
        out_specs=pl.BlockSpec((1, MAXN, ns), lambda b: (b, 0, 0)),
        out_shape=jax.ShapeDtypeStruct((NB, MAXN, ns), jnp.float32),
    )(batch.reshape(n // 128, 128), outp)
    return dense


# speedup vs baseline: 3.9439x; 1.2434x over previous
import functools

import jax
import jax.numpy as jnp
from jax import lax
from jax.experimental import pallas as pl
from jax.experimental.pallas import tpu as pltpu
from jax.experimental.pallas import tpu_sc as plsc

NB = 16
MAXN = 2048
CHUNK = 64


def _ln(x):
    mu = jnp.mean(x, axis=-1, keepdims=True)
    var = jnp.var(x, axis=-1, keepdims=True)
    return (x - mu) * lax.rsqrt(var + 1e-5)


def _node_embed_body(ns_ref, wv_ref, bv_ref, wa_ref, wc_ref,
                     h_ref, p_ref, q_ref):
    h = jnp.dot(_ln(ns_ref[...]), wv_ref[...],
                preferred_element_type=jnp.float32) + bv_ref[...]
    h_ref[...] = h
    p_ref[...] = jnp.dot(h, wa_ref[...], preferred_element_type=jnp.float32)
    q_ref[...] = jnp.dot(h, wc_ref[...], preferred_element_type=jnp.float32)


def _edge_embed_body(es_ref, we_ref, be_ref, wb_ref, bm_ref, r_ref):
    e = jnp.dot(_ln(es_ref[...]), we_ref[...],
                preferred_element_type=jnp.float32) + be_ref[...]
    for l in range(wb_ref.shape[0]):
        r_ref[l] = jnp.dot(e, wb_ref[l],
                           preferred_element_type=jnp.float32) + bm_ref[l]


def _update_body(h_ref, agg_ref, deg_ref, wa_ref, wc_ref,
                 h_ref_out, p_ref, q_ref):
    deg = jnp.maximum(deg_ref[:, 0], 1.0)
    h = _ln(h_ref[...] + agg_ref[...] / deg[:, None])
    h_ref_out[...] = h
    p_ref[...] = jnp.dot(h, wa_ref[...], preferred_element_type=jnp.float32)
    q_ref[...] = jnp.dot(h, wc_ref[...], preferred_element_type=jnp.float32)


def _final_body(h_ref, agg_ref, deg_ref, wo_ref, bo_ref, out_ref):
    deg = jnp.maximum(deg_ref[:, 0], 1.0)
    h = _ln(h_ref[...] + agg_ref[...] / deg[:, None])
    out_ref[...] = jnp.dot(_ln(h), wo_ref[...],
                           preferred_element_type=jnp.float32) + bo_ref[...]


def _pad_body(batch_ref, out_ref, dense_ref):
    b = pl.program_id(0)
    bm = batch_ref[...]
    ptr = jnp.sum(jnp.where(bm < b, 1, 0))
    cnt = jnp.sum(jnp.where(bm == b, 1, 0))
    rows = out_ref[pl.ds(ptr, MAXN), :]
    keep = lax.broadcasted_iota(jnp.int32, (MAXN, 1), 0) < cnt
    dense_ref[0] = jnp.where(keep, rows, 0.0)


def _sc_prep_body(src_hbm, dst_hbm, zeros_hbm,
                  src_c, dst_c, eid_c, cnts, deg_out,
                  acc, sv, dv, dlv, sbuf, dbuf, ebuf, ones_v, cbuf, sem):
    c = lax.axis_index("c")
    s = lax.axis_index("s")
    half = acc.shape[0] - 8
    rows_per_tile = half // 16
    for j in range(rows_per_tile // CHUNK):
        pltpu.sync_copy(zeros_hbm,
                        acc.at[pl.ds(s * rows_per_tile + j * CHUNK, CHUNK)])
    one = jnp.ones((16,), jnp.float32)
    zero_i = jnp.zeros((16,), jnp.int32)
    for i in range(CHUNK):
        ones_v[i, pl.ds(0, 16)] = one
    trash_dst = jnp.full((16,), 0, jnp.int32) + (1 - c) * half
    per_tile = src_hbm.shape[0] // 16
    for i in range(per_tile // 16):
        sl = pl.ds(i * 16, 16)
        sbuf[sl] = zero_i
        dbuf[sl] = trash_dst
        ebuf[sl] = zero_i
    plsc.subcore_barrier()

    base0 = s * per_tile
    lo = c * half

    def body(g, off):
        base = base0 + g * CHUNK
        pltpu.sync_copy(src_hbm.at[pl.ds(base, CHUNK)], sv)
        pltpu.sync_copy(dst_hbm.at[pl.ds(base, CHUNK)], dv)
        for j in range(CHUNK // 16):
            sl = pl.ds(j * 16, 16)
            d = dv[sl]
            dl = d - lo
            ok = (dl >= 0) & (dl < half)
            dlv[sl] = jnp.where(ok, dl, half)
            eid = base + j * 16 + lax.iota(jnp.int32, 16)
            ok_i = ok.astype(jnp.int32)
            pos = plsc.cumsum(ok_i) - 1 + off
            idx = jnp.where(ok, pos, per_tile)
            plsc.store_scatter(sbuf, [idx], sv[sl])
            plsc.store_scatter(dbuf, [idx], d)
            plsc.store_scatter(ebuf, [idx], eid)
            off = off + jnp.sum(ok_i)
        pltpu.sync_copy(ones_v, acc.at[dlv], add=True)
        return off

    n_valid = lax.fori_loop(0, per_tile // CHUNK, body, 0)

    flat0 = c * src_hbm.shape[0] + base0
    pltpu.sync_copy(sbuf.at[pl.ds(0, per_tile)], src_c.at[pl.ds(flat0, per_tile)])
    pltpu.sync_copy(dbuf.at[pl.ds(0, per_tile)], dst_c.at[pl.ds(flat0, per_tile)])
    pltpu.sync_copy(ebuf.at[pl.ds(0, per_tile)], eid_c.at[pl.ds(flat0, per_tile)])
    cbuf[...] = jnp.zeros((16,), jnp.int32) + n_valid
    pltpu.sync_copy(cbuf, cnts.at[pl.ds(c * 256 + s * 16, 16)])

    plsc.subcore_barrier()
    pltpu.sync_copy(acc.at[pl.ds(s * rows_per_tile, rows_per_tile)],
                    deg_out.at[pl.ds(c * half + s * rows_per_tile,
                                     rows_per_tile)])


def _sc_layer_body(p_hbm, q_hbm, r_hbm, src_c, dst_c, eid_c, cnts, zeros_hbm,
                   agg_out, acc,
                   slist, dlist, elist, cntv,
                   dstl0, pbuf0, qbuf0, rbuf0,
                   dstl1, pbuf1, qbuf1, rbuf1,
                   gsem0, gsem1, *, layer):
    c = lax.axis_index("c")
    s = lax.axis_index("s")
    half = acc.shape[0] - 8
    rows_per_tile = half // 16
    e = src_c.shape[0] // 2
    per_tile = e // 16
    flat0 = c * e + s * per_tile
    for j in range(rows_per_tile // CHUNK):
        pltpu.sync_copy(zeros_hbm,
                        acc.at[pl.ds(s * rows_per_tile + j * CHUNK, CHUNK)])
    pltpu.sync_copy(src_c.at[pl.ds(flat0, per_tile)], slist)
    pltpu.sync_copy(dst_c.at[pl.ds(flat0, per_tile)], dlist)
    pltpu.sync_copy(eid_c.at[pl.ds(flat0, per_tile)], elist)
    pltpu.sync_copy(cnts.at[pl.ds(c * 256 + s * 16, 16)], cntv)
    plsc.subcore_barrier()

    n_valid = jnp.max(cntv[...])
    nch = (n_valid + CHUNK - 1) // CHUNK

    bufs = ((dstl0, pbuf0, qbuf0, rbuf0, gsem0),
            (dstl1, pbuf1, qbuf1, rbuf1, gsem1))

    def issue(g, bb):
        _, pbuf, qbuf, rbuf, gsem = bb
        sl = pl.ds(g * CHUNK, CHUNK)
        pltpu.async_copy(p_hbm.at[slist.at[sl]], pbuf, gsem)
        pltpu.async_copy(q_hbm.at[dlist.at[sl]], qbuf, gsem)
        pltpu.async_copy(r_hbm.at[layer].at[elist.at[sl]], rbuf, gsem)

    def drain(bb):
        _, pbuf, qbuf, rbuf, gsem = bb
        sl = pl.ds(0, CHUNK)
        pltpu.make_async_copy(p_hbm.at[slist.at[sl]], pbuf, gsem).wait()
        pltpu.make_async_copy(q_hbm.at[dlist.at[sl]], qbuf, gsem).wait()
        pltpu.make_async_copy(p_hbm.at[elist.at[sl]], rbuf, gsem).wait()

    @pl.when(0 < nch)
    def _():
        issue(0, bufs[0])

    @pl.loop(0, (nch + 1) // 2)
    def _outer(gg):
        for b in range(2):
            dstl, pbuf, qbuf, rbuf, gsem = bufs[b]
            g = gg * 2 + b

            @pl.when(g < nch)
            def _():
                drain(bufs[b])

                @pl.when(g + 1 < nch)
                def _():
                    issue(g + 1, bufs[1 - b])

                @plsc.parallel_loop(0, CHUNK, unroll=4)
                def _rows(rr):
                    for j in range(8):
                        sl = pl.ds(j * 16, 16)
                        m = pbuf[rr, sl] + qbuf[rr, sl] + rbuf[rr, sl]
                        pbuf[rr, sl] = jnp.maximum(m, 0.0)

                lo = c * half
                for j in range(CHUNK // 16):
                    sl = pl.ds(j * 16, 16)
                    d = dlist[pl.ds(g * CHUNK + j * 16, 16)] - lo
                    ok = (d >= 0) & (d < half)
                    dstl[sl] = jnp.where(ok, d, half)
                pltpu.sync_copy(pbuf, acc.at[dstl], add=True)

    plsc.subcore_barrier()
    pltpu.sync_copy(acc.at[pl.ds(s * rows_per_tile, rows_per_tile)],
                    agg_out.at[pl.ds(c * half + s * rows_per_tile,
                                     rows_per_tile)])


def kernel(node_s, edge_s, edge_index, batch, Wv, bv, We, be, Wm, bm, Wout, bout):
    n, ns_in = node_s.shape
    e, es_in = edge_s.shape
    layers, msg_in, ns = Wm.shape
    es = We.shape[1]

    src = edge_index[0]
    dst = edge_index[1]
    zeros = jnp.zeros((CHUNK, ns), jnp.float32)
    zeros16 = jnp.zeros((CHUNK, 16), jnp.float32)

    bn = 1024
    wa = [Wm[l, :ns, :] for l in range(layers)]
    wc = [Wm[l, ns + es:, :] for l in range(layers)]
    wb = Wm[:, ns:ns + es, :]

    h, p, q = pl.pallas_call(
        _node_embed_body,
        grid=(n // bn,),
        in_specs=[
            pl.BlockSpec((bn, ns_in), lambda i: (i, 0)),
            pl.BlockSpec((ns_in, ns), lambda i: (0, 0)),
            pl.BlockSpec((1, ns), lambda i: (0, 0)),
            pl.BlockSpec((ns, ns), lambda i: (0, 0)),
            pl.BlockSpec((ns, ns), lambda i: (0, 0)),
        ],
        out_specs=[
            pl.BlockSpec((bn, ns), lambda i: (i, 0)),
            pl.BlockSpec((bn, ns), lambda i: (i, 0)),
            pl.BlockSpec((bn, ns), lambda i: (i, 0)),
        ],
        out_shape=[
            jax.ShapeDtypeStruct((n, ns), jnp.float32),
            jax.ShapeDtypeStruct((n, ns), jnp.float32),
            jax.ShapeDtypeStruct((n, ns), jnp.float32),
        ],
    )(node_s, Wv, bv.reshape(1, ns), wa[0], wc[0])

    be_blk = 2048
    r = pl.pallas_call(
        _edge_embed_body,
        grid=(e // be_blk,),
        in_specs=[
            pl.BlockSpec((be_blk, es_in), lambda i: (i, 0)),
            pl.BlockSpec((es_in, es), lambda i: (0, 0)),
            pl.BlockSpec((1, es), lambda i: (0, 0)),
            pl.BlockSpec((layers, es, ns), lambda i: (0, 0, 0)),
            pl.BlockSpec((layers, 1, ns), lambda i: (0, 0, 0)),
        ],
        out_specs=pl.BlockSpec((layers, be_blk, ns), lambda i: (0, i, 0)),
        out_shape=jax.ShapeDtypeStruct((layers, e, ns), jnp.float32),
    )(edge_s, We, be.reshape(1, es), wb, bm.reshape(layers, 1, ns))

    mesh = plsc.VectorSubcoreMesh(core_axis_name="c", subcore_axis_name="s")
    per_tile = e // 16
    src_c, dst_c, eid_c, cnts, deg = pl.kernel(
        _sc_prep_body,
        out_type=[
            jax.ShapeDtypeStruct((2 * e,), jnp.int32),
            jax.ShapeDtypeStruct((2 * e,), jnp.int32),
            jax.ShapeDtypeStruct((2 * e,), jnp.int32),
            jax.ShapeDtypeStruct((512,), jnp.int32),
            jax.ShapeDtypeStruct((n, 16), jnp.float32),
        ],
        mesh=mesh,
        compiler_params=pltpu.CompilerParams(needs_layout_passes=False),
        scratch_types=[
            pltpu.VMEM_SHARED((n // 2 + 8, 16), jnp.float32),
            pltpu.VMEM((CHUNK,), jnp.int32),
            pltpu.VMEM((CHUNK,), jnp.int32),
            pltpu.VMEM((CHUNK,), jnp.int32),
            pltpu.VMEM((per_tile + 16,), jnp.int32),
            pltpu.VMEM((per_tile + 16,), jnp.int32),
            pltpu.VMEM((per_tile + 16,), jnp.int32),
            pltpu.VMEM((CHUNK, 16), jnp.float32),
            pltpu.VMEM((16,), jnp.int32),
            pltpu.SemaphoreType.DMA,
        ],
    )(src, dst, zeros16)

    for l in range(layers):
        agg = pl.kernel(
            functools.partial(_sc_layer_body, layer=l),
            out_type=jax.ShapeDtypeStruct((n, ns), jnp.float32),
            mesh=mesh,
            compiler_params=pltpu.CompilerParams(needs_layout_passes=False),
            scratch_types=[
                pltpu.VMEM_SHARED((n // 2 + 8, ns), jnp.float32),
                pltpu.VMEM((e // 16,), jnp.int32),
                pltpu.VMEM((e // 16,), jnp.int32),
                pltpu.VMEM((e // 16,), jnp.int32),
                pltpu.VMEM((16,), jnp.int32),
                pltpu.VMEM((CHUNK,), jnp.int32),
                pltpu.VMEM((CHUNK, ns), jnp.float32),
                pltpu.VMEM((CHUNK, ns), jnp.float32),
                pltpu.VMEM((CHUNK, ns), jnp.float32),
                pltpu.VMEM((CHUNK,), jnp.int32),
                pltpu.VMEM((CHUNK, ns), jnp.float32),
                pltpu.VMEM((CHUNK, ns), jnp.float32),
                pltpu.VMEM((CHUNK, ns), jnp.float32),
                pltpu.SemaphoreType.DMA,
                pltpu.SemaphoreType.DMA,
            ],
        )(p, q, r, src_c, dst_c, eid_c, cnts, zeros)

        if l < layers - 1:
            h, p, q = pl.pallas_call(
                _update_body,
                grid=(n // bn,),
                in_specs=[
                    pl.BlockSpec((bn, ns), lambda i: (i, 0)),
                    pl.BlockSpec((bn, ns), lambda i: (i, 0)),
                    pl.BlockSpec((bn, 16), lambda i: (i, 0)),
                    pl.BlockSpec((ns, ns), lambda i: (0, 0)),
                    pl.BlockSpec((ns, ns), lambda i: (0, 0)),
                ],
                out_specs=[
                    pl.BlockSpec((bn, ns), lambda i: (i, 0)),
                    pl.BlockSpec((bn, ns), lambda i: (i, 0)),
                    pl.BlockSpec((bn, ns), lambda i: (i, 0)),
                ],
                out_shape=[
                    jax.ShapeDtypeStruct((n, ns), jnp.float32),
                    jax.ShapeDtypeStruct((n, ns), jnp.float32),
                    jax.ShapeDtypeStruct((n, ns), jnp.float32),
                ],
            )(h, agg, deg, wa[l + 1], wc[l + 1])
        else:
            out = pl.pallas_call(
                _final_body,
                grid=(n // bn,),
                in_specs=[
                    pl.BlockSpec((bn, ns), lambda i: (i, 0)),
                    pl.BlockSpec((bn, ns), lambda i: (i, 0)),
                    pl.BlockSpec((bn, 16), lambda i: (i, 0)),
                    pl.BlockSpec((ns, ns), lambda i: (0, 0)),
                    pl.BlockSpec((1, ns), lambda i: (0, 0)),
                ],
                out_specs=pl.BlockSpec((bn, ns), lambda i: (i, 0)),
                out_shape=jax.ShapeDtypeStruct((n, ns), jnp.float32),
            )(h, agg, deg, Wout, bout.reshape(1, ns))

    outp = jnp.pad(out, ((0, MAXN), (0, 0)))
    dense = pl.pallas_call(
        _pad_body,
        grid=(NB,),
        in_specs=[
            pl.BlockSpec((n // 128, 128), lambda b: (0, 0)),
            pl.BlockSpec((n + MAXN, ns), lambda b: (0, 0)),
        ],
        out_specs=pl.BlockSpec((1, MAXN, ns), lambda b: (b, 0, 0)),
        out_shape=jax.ShapeDtypeStruct((NB, MAXN, ns), jnp.float32),
    )(batch.reshape(n // 128, 128), outp)
    return dense
